# baseline (speedup 1.0000x reference)
import functools

import jax
import jax.numpy as jnp
from jax import lax
from jax.experimental import pallas as pl
from jax.experimental.pallas import tpu as pltpu
from jax.experimental.pallas import tpu_sc as plsc

_NC, _NS, _L = 2, 16, 16
_NW = _NC * _NS

_N, _C, _R = 4, 4, 4096
_TAB = _C * _R
_VOX = 128 * 128 * 128
_NP = _NW // _N
_VW = _VOX // _NP
_CH = 8192
_T = _VW // _CH
_NV = _CH // _L


def _tf_body(x_hbm, tf_hbm, out_hbm, tab, xbuf, obuf, in_sem, out_sem):
    wid = lax.axis_index("s") * _NC + lax.axis_index("c")
    n = wid % _N
    part = wid // _N

    pltpu.sync_copy(tf_hbm.at[pl.ds(n * _TAB, _TAB)], tab)

    iota = lax.iota(jnp.int32, _L)
    rnd = jnp.full((_L,), 0x8000, dtype=jnp.int32)
    himask = jnp.full((_L,), -0x10000, dtype=jnp.int32)

    def pack_body(k, _):
        y0 = tab[pl.ds(k * _L, _L)]
        y1 = plsc.load_gather(tab, [jnp.minimum(iota + (k * _L + 1), _TAB - 1)])
        d = y1 - y0
        y0b = (plsc.bitcast(y0, jnp.int32) + rnd) & himask
        db = lax.shift_right_logical(plsc.bitcast(d, jnp.int32) + rnd, 16)
        tab[pl.ds(k * _L, _L)] = plsc.bitcast(y0b | db, jnp.float32)
        return 0

    lax.fori_loop(0, _TAB // _L, pack_body, 0)

    def x_slice(t):
        return x_hbm.at[pl.ds(n * _VOX + part * _VW + t * _CH, _CH)]

    pltpu.async_copy(x_slice(0), xbuf.at[pl.ds(0, _CH)], in_sem)

    def chunk_body(t, _):
        s = t % 2
        pltpu.make_async_copy(x_slice(t), xbuf.at[pl.ds(s * _CH, _CH)], in_sem).wait()

        @pl.when(t + 1 < _T)
        def _prefetch():
            s2 = (t + 1) % 2
            pltpu.async_copy(x_slice(t + 1), xbuf.at[pl.ds(s2 * _CH, _CH)], in_sem)

        @pl.when(t >= 2)
        def _drain():
            pltpu.make_async_copy(
                x_hbm.at[pl.ds(0, _C * _CH)],
                obuf.at[pl.ds(s * _C * _CH, _C * _CH)],
                out_sem,
            ).wait()

        xb = s * _CH
        ob = s * (_C * _CH)

        @plsc.parallel_loop(0, _NV, 1, unroll=8)
        def vreg_body(k):
            xv = xbuf[pl.ds(xb + k * _L, _L)]
            t_ = xv * float(_R - 1)
            i = t_.astype(jnp.int32)
            f = t_ - i.astype(jnp.float32)
            for c in range(_C):
                w = plsc.bitcast(plsc.load_gather(tab, [i + c * _R]), jnp.int32)
                y0 = plsc.bitcast(w & himask, jnp.float32)
                d = plsc.bitcast(lax.shift_left(w, 16), jnp.float32)
                obuf[pl.ds(ob + c * _CH + k * _L, _L)] = y0 + f * d

        for c in range(_C):
            ooff = (n * _C + c) * _VOX + part * _VW + t * _CH
            pltpu.async_copy(
                obuf.at[pl.ds(ob + c * _CH, _CH)],
                out_hbm.at[pl.ds(ooff, _CH)],
                out_sem,
            )
        return 0

    lax.fori_loop(0, _T, chunk_body, 0)

    pltpu.make_async_copy(x_hbm.at[pl.ds(0, 2 * _C * _CH)], obuf, out_sem).wait()


_tf_apply = functools.partial(
    pl.kernel,
    out_type=jax.ShapeDtypeStruct((_N * _C * _VOX,), jnp.float32),
    mesh=plsc.VectorSubcoreMesh(core_axis_name="c", subcore_axis_name="s"),
    compiler_params=pltpu.CompilerParams(needs_layout_passes=False),
    scratch_types=[
        pltpu.VMEM((_TAB,), jnp.float32),
        pltpu.VMEM((2 * _CH,), jnp.float32),
        pltpu.VMEM((2 * _C * _CH,), jnp.float32),
        pltpu.SemaphoreType.DMA,
        pltpu.SemaphoreType.DMA,
    ],
)(_tf_body)


def kernel(x, tf):
    x_flat = x.reshape(-1).astype(jnp.float32)
    tf_flat = tf.reshape(-1).astype(jnp.float32)
    out = _tf_apply(x_flat, tf_flat)
    return out.reshape(_N, _C, 128, 128, 128).astype(x.dtype)


# speedup vs baseline: 1.0156x; 1.0156x over previous
import functools

import jax
import jax.numpy as jnp
from jax import lax
from jax.experimental import pallas as pl
from jax.experimental.pallas import tpu as pltpu
from jax.experimental.pallas import tpu_sc as plsc

_NC, _NS, _L = 2, 16, 16
_NW = _NC * _NS

_N, _C, _R = 4, 4, 4096
_TAB = _C * _R
_VOX = 128 * 128 * 128
_NP = _NW // _N
_VW = _VOX // _NP
_CH = 4096
_T = _VW // _CH
_NV = _CH // _L


def _tf_body(x_hbm, tf_hbm, out_hbm, tab, xbuf, obuf, in_sem, out_sem):
    wid = lax.axis_index("s") * _NC + lax.axis_index("c")
    n = wid % _N
    part = wid // _N

    pltpu.sync_copy(tf_hbm.at[pl.ds(n * _TAB, _TAB)], tab)

    iota = lax.iota(jnp.int32, _L)
    rnd = jnp.full((_L,), 0x8000, dtype=jnp.int32)
    himask = jnp.full((_L,), -0x10000, dtype=jnp.int32)

    def pack_body(k, _):
        y0 = tab[pl.ds(k * _L, _L)]
        y1 = plsc.load_gather(tab, [jnp.minimum(iota + (k * _L + 1), _TAB - 1)])
        d = y1 - y0
        y0b = (plsc.bitcast(y0, jnp.int32) + rnd) & himask
        db = lax.shift_right_logical(plsc.bitcast(d, jnp.int32) + rnd, 16)
        tab[pl.ds(k * _L, _L)] = plsc.bitcast(y0b | db, jnp.float32)
        return 0

    lax.fori_loop(0, _TAB // _L, pack_body, 0)

    def x_slice(t):
        return x_hbm.at[pl.ds(n * _VOX + part * _VW + t * _CH, _CH)]

    pltpu.async_copy(x_slice(0), xbuf.at[pl.ds(0, _CH)], in_sem)

    def chunk_body(t, _):
        s = t % 2
        pltpu.make_async_copy(x_slice(t), xbuf.at[pl.ds(s * _CH, _CH)], in_sem).wait()

        @pl.when(t + 1 < _T)
        def _prefetch():
            s2 = (t + 1) % 2
            pltpu.async_copy(x_slice(t + 1), xbuf.at[pl.ds(s2 * _CH, _CH)], in_sem)

        @pl.when(t >= 2)
        def _drain():
            pltpu.make_async_copy(
                x_hbm.at[pl.ds(0, _C * _CH)],
                obuf.at[pl.ds(s * _C * _CH, _C * _CH)],
                out_sem,
            ).wait()

        xb = s * _CH
        ob = s * (_C * _CH)

        @plsc.parallel_loop(0, _NV, 1, unroll=8)
        def vreg_body(k):
            xv = xbuf[pl.ds(xb + k * _L, _L)]
            t_ = xv * float(_R - 1)
            i = t_.astype(jnp.int32)
            f = t_ - i.astype(jnp.float32)
            for c in range(_C):
                w = plsc.bitcast(plsc.load_gather(tab, [i + c * _R]), jnp.int32)
                y0 = plsc.bitcast(w & himask, jnp.float32)
                d = plsc.bitcast(lax.shift_left(w, 16), jnp.float32)
                obuf[pl.ds(ob + c * _CH + k * _L, _L)] = y0 + f * d

        for c in range(_C):
            ooff = (n * _C + c) * _VOX + part * _VW + t * _CH
            pltpu.async_copy(
                obuf.at[pl.ds(ob + c * _CH, _CH)],
                out_hbm.at[pl.ds(ooff, _CH)],
                out_sem,
            )
        return 0

    lax.fori_loop(0, _T, chunk_body, 0)

    pltpu.make_async_copy(x_hbm.at[pl.ds(0, 2 * _C * _CH)], obuf, out_sem).wait()


_tf_apply = functools.partial(
    pl.kernel,
    out_type=jax.ShapeDtypeStruct((_N * _C * _VOX,), jnp.float32),
    mesh=plsc.VectorSubcoreMesh(core_axis_name="c", subcore_axis_name="s"),
    compiler_params=pltpu.CompilerParams(needs_layout_passes=False),
    scratch_types=[
        pltpu.VMEM((_TAB,), jnp.float32),
        pltpu.VMEM((2 * _CH,), jnp.float32),
        pltpu.VMEM((2 * _C * _CH,), jnp.float32),
        pltpu.SemaphoreType.DMA,
        pltpu.SemaphoreType.DMA,
    ],
)(_tf_body)


def kernel(x, tf):
    x_flat = x.reshape(-1).astype(jnp.float32)
    tf_flat = tf.reshape(-1).astype(jnp.float32)
    out = _tf_apply(x_flat, tf_flat)
    return out.reshape(_N, _C, 128, 128, 128).astype(x.dtype)
